# baseline (speedup 1.0000x reference)
import numpy as np

import jax
import jax.numpy as jnp
from jax import lax
from jax.experimental import pallas as pl
from jax.experimental.pallas import tpu as pltpu
from jax.experimental.pallas import tpu_sc as plsc

B = 128
V = 100000

_NEG_INF = float("-inf")


def _gumbel_constant() -> np.ndarray:
    def rotl(x, d):
        return (x << np.uint32(d)) | (x >> np.uint32(32 - d))

    rot = [np.uint32([13, 15, 26, 6]), np.uint32([17, 29, 16, 24])]
    k1, k2 = np.uint32(0), np.uint32(42)
    ks = [k1, k2, np.uint32(k1 ^ k2 ^ np.uint32(0x1BD11BDA))]
    x = [np.uint32(0) + ks[0], np.arange(B * V, dtype=np.uint32) + ks[1]]
    with np.errstate(over="ignore"):
        for i in range(5):
            for r in rot[i % 2]:
                x[0] = x[0] + x[1]
                x[1] = rotl(x[1], int(r))
                x[1] = x[0] ^ x[1]
            x[0] = x[0] + ks[(i + 1) % 3]
            x[1] = x[1] + ks[(i + 2) % 3] + np.uint32(i + 1)
    bits = x[0] ^ x[1]
    float_bits = (bits >> np.uint32(9)) | np.uint32(0x3F800000)
    floats = float_bits.view(np.float32) - np.float32(1.0)
    tiny = np.float32(np.finfo(np.float32).tiny)
    u = np.maximum(tiny, floats * (np.float32(1.0) - tiny) + tiny)
    g = (-np.log(-np.log(u))).astype(np.float32).reshape(B, V)
    g[:, 0] = _NEG_INF
    return g


_GUMBEL = _gumbel_constant()
_GUMBEL_FLAT = _GUMBEL.reshape(-1)


NC = 2
NS = 16
NW = NC * NS
RPW = B // NW
CH = 20000
NCH = V // CH
LPC = CH // 16


def _sc_body(x_hbm, g_hbm, out_hbm, xb0, xb1, gb0, gb1, res_v,
             sx0, sx1, sg0, sg1):
    wid = lax.axis_index("s") * NC + lax.axis_index("c")
    row0 = wid * RPW
    xbufs, gbufs = (xb0, xb1), (gb0, gb1)
    sxs, sgs = (sx0, sx1), (sg0, sg1)
    iota = lax.broadcasted_iota(jnp.int32, (16,), 0)

    pairs = [(j, c) for j in range(RPW) for c in range(NCH)]

    def issue(t):
        j, c = pairs[t]
        slot = t % 2
        off = (row0 + j) * V + c * CH
        dx = pltpu.async_copy(x_hbm.at[pl.ds(off, CH)], xbufs[slot], sxs[slot])
        dg = pltpu.async_copy(g_hbm.at[pl.ds(off, CH)], gbufs[slot], sgs[slot])
        return dx, dg

    m = jnp.full((16,), _NEG_INF, jnp.float32)
    mi = jnp.zeros((16,), jnp.int32)
    res = jnp.zeros((16,), jnp.int32)

    pending = {0: issue(0)}
    for t in range(len(pairs)):
        j, c = pairs[t]
        slot = t % 2
        dx, dg = pending.pop(t)
        dx.wait()
        dg.wait()
        if t + 1 < len(pairs):
            pending[t + 1] = issue(t + 1)

        xb, gb = xbufs[slot], gbufs[slot]
        base = c * CH

        @plsc.parallel_loop(0, LPC, step=1, unroll=8, carry=(m, mi))
        def _scan(k, carry):
            mm, ii = carry
            s = xb[pl.ds(k * 16, 16)] + gb[pl.ds(k * 16, 16)]
            idx = iota + (base + k * 16)
            upd = s > mm
            return jnp.where(upd, s, mm), jnp.where(upd, idx, ii)

        m, mi = _scan

        if c == NCH - 1:
            mv = jnp.max(m)
            cand = jnp.where(m == mv, mi, jnp.int32(V))
            res = jnp.where(iota == j, jnp.min(cand), res)
            m = jnp.full((16,), _NEG_INF, jnp.float32)
            mi = jnp.zeros((16,), jnp.int32)

    res_v[...] = res
    pltpu.sync_copy(res_v, out_hbm.at[wid])


def _sc_sample(x_flat):
    out = pl.kernel(
        _sc_body,
        out_type=jax.ShapeDtypeStruct((NW, 16), jnp.int32),
        mesh=plsc.VectorSubcoreMesh(core_axis_name="c", subcore_axis_name="s"),
        compiler_params=pltpu.CompilerParams(needs_layout_passes=False),
        scratch_types=[
            pltpu.VMEM((CH,), jnp.float32),
            pltpu.VMEM((CH,), jnp.float32),
            pltpu.VMEM((CH,), jnp.float32),
            pltpu.VMEM((CH,), jnp.float32),
            pltpu.VMEM((16,), jnp.int32),
            pltpu.SemaphoreType.DMA,
            pltpu.SemaphoreType.DMA,
            pltpu.SemaphoreType.DMA,
            pltpu.SemaphoreType.DMA,
        ],
    )(x_flat, _GUMBEL_FLAT)
    return out[:, :RPW].reshape(B, 1)


def kernel(decoder_out):
    return _sc_sample(decoder_out.reshape(-1))


# speedup vs baseline: 1.0002x; 1.0002x over previous
import numpy as np

import jax
import jax.numpy as jnp
from jax import lax
from jax.experimental import pallas as pl
from jax.experimental.pallas import tpu as pltpu
from jax.experimental.pallas import tpu_sc as plsc

B = 128
V = 100000

_NEG_INF = float("-inf")


def _gumbel_constant() -> np.ndarray:
    def rotl(x, d):
        return (x << np.uint32(d)) | (x >> np.uint32(32 - d))

    rot = [np.uint32([13, 15, 26, 6]), np.uint32([17, 29, 16, 24])]
    k1, k2 = np.uint32(0), np.uint32(42)
    ks = [k1, k2, np.uint32(k1 ^ k2 ^ np.uint32(0x1BD11BDA))]
    x = [np.uint32(0) + ks[0], np.arange(B * V, dtype=np.uint32) + ks[1]]
    with np.errstate(over="ignore"):
        for i in range(5):
            for r in rot[i % 2]:
                x[0] = x[0] + x[1]
                x[1] = rotl(x[1], int(r))
                x[1] = x[0] ^ x[1]
            x[0] = x[0] + ks[(i + 1) % 3]
            x[1] = x[1] + ks[(i + 2) % 3] + np.uint32(i + 1)
    bits = x[0] ^ x[1]
    float_bits = (bits >> np.uint32(9)) | np.uint32(0x3F800000)
    floats = float_bits.view(np.float32) - np.float32(1.0)
    tiny = np.float32(np.finfo(np.float32).tiny)
    u = np.maximum(tiny, floats * (np.float32(1.0) - tiny) + tiny)
    g = (-np.log(-np.log(u))).astype(np.float32).reshape(B, V)
    g[:, 0] = _NEG_INF
    return g


_GUMBEL = _gumbel_constant()
_GUMBEL_FLAT = _GUMBEL.reshape(-1)


NC = 2
NS = 16
NW = NC * NS
RPW = B // NW
CH = 20000
NCH = V // CH
LPC = CH // 16


def _sc_body(x_hbm, g_hbm, out_hbm, xb0, xb1, gb0, gb1, res_v,
             sx0, sx1, sg0, sg1):
    wid = lax.axis_index("s") * NC + lax.axis_index("c")
    row0 = wid * RPW
    xbufs, gbufs = (xb0, xb1), (gb0, gb1)
    sxs, sgs = (sx0, sx1), (sg0, sg1)
    iota = lax.broadcasted_iota(jnp.int32, (16,), 0)

    pairs = [(j, c) for j in range(RPW) for c in range(NCH)]

    def issue(t):
        j, c = pairs[t]
        slot = t % 2
        off = (row0 + j) * V + c * CH
        dx = pltpu.async_copy(x_hbm.at[pl.ds(off, CH)], xbufs[slot], sxs[slot])
        dg = pltpu.async_copy(g_hbm.at[pl.ds(off, CH)], gbufs[slot], sgs[slot])
        return dx, dg

    NACC = 4
    def fresh():
        return (tuple(jnp.full((16,), _NEG_INF, jnp.float32) for _ in range(NACC))
                + tuple(jnp.zeros((16,), jnp.int32) for _ in range(NACC)))

    acc = fresh()
    res = jnp.zeros((16,), jnp.int32)

    pending = {0: issue(0)}
    for t in range(len(pairs)):
        j, c = pairs[t]
        slot = t % 2
        dx, dg = pending.pop(t)
        dx.wait()
        dg.wait()
        if t + 1 < len(pairs):
            pending[t + 1] = issue(t + 1)

        xb, gb = xbufs[slot], gbufs[slot]
        base = c * CH

        @plsc.parallel_loop(0, LPC // NACC, step=1, unroll=4, carry=acc)
        def _scan(k, carry):
            ms, iis = list(carry[:NACC]), list(carry[NACC:])
            for a in range(NACC):
                o = k * (16 * NACC) + a * 16
                s = xb[pl.ds(o, 16)] + gb[pl.ds(o, 16)]
                idx = iota + (base + o)
                upd = s > ms[a]
                ms[a] = jnp.where(upd, s, ms[a])
                iis[a] = jnp.where(upd, idx, iis[a])
            return tuple(ms) + tuple(iis)

        acc = list(_scan)
        for a in range(LPC % NACC):
            o = (LPC // NACC) * (16 * NACC) + a * 16
            s = xb[pl.ds(o, 16)] + gb[pl.ds(o, 16)]
            idx = iota + (base + o)
            upd = s > acc[a]
            acc[NACC + a] = jnp.where(upd, idx, acc[NACC + a])
            acc[a] = jnp.where(upd, s, acc[a])
        acc = tuple(acc)

        if c == NCH - 1:
            ms, iis = acc[:NACC], acc[NACC:]
            m, mi = ms[0], iis[0]
            for a in range(1, NACC):
                take = (ms[a] > m) | ((ms[a] == m) & (iis[a] < mi))
                m = jnp.where(take, ms[a], m)
                mi = jnp.where(take, iis[a], mi)
            mv = jnp.max(m)
            cand = jnp.where(m == mv, mi, jnp.int32(V))
            res = jnp.where(iota == j, jnp.min(cand), res)
            acc = fresh()

    res_v[...] = res
    pltpu.sync_copy(res_v, out_hbm.at[wid])


def _sc_sample(x_flat):
    out = pl.kernel(
        _sc_body,
        out_type=jax.ShapeDtypeStruct((NW, 16), jnp.int32),
        mesh=plsc.VectorSubcoreMesh(core_axis_name="c", subcore_axis_name="s"),
        compiler_params=pltpu.CompilerParams(needs_layout_passes=False),
        scratch_types=[
            pltpu.VMEM((CH,), jnp.float32),
            pltpu.VMEM((CH,), jnp.float32),
            pltpu.VMEM((CH,), jnp.float32),
            pltpu.VMEM((CH,), jnp.float32),
            pltpu.VMEM((16,), jnp.int32),
            pltpu.SemaphoreType.DMA,
            pltpu.SemaphoreType.DMA,
            pltpu.SemaphoreType.DMA,
            pltpu.SemaphoreType.DMA,
        ],
    )(x_flat, _GUMBEL_FLAT)
    return out[:, :RPW].reshape(B, 1)


def kernel(decoder_out):
    return _sc_sample(decoder_out.reshape(-1))


# speedup vs baseline: 2.5837x; 2.5830x over previous
import numpy as np

import jax
import jax.numpy as jnp
from jax.experimental import pallas as pl
from jax.experimental.pallas import tpu as pltpu

B = 128
V = 100000
VC = 8192
NBLK = (V + VC - 1) // VC

_NEG_INF = float("-inf")


def _gumbel_constant() -> np.ndarray:
    def rotl(x, d):
        return (x << np.uint32(d)) | (x >> np.uint32(32 - d))

    rot = [np.uint32([13, 15, 26, 6]), np.uint32([17, 29, 16, 24])]
    k1, k2 = np.uint32(0), np.uint32(42)
    ks = [k1, k2, np.uint32(k1 ^ k2 ^ np.uint32(0x1BD11BDA))]
    x = [np.uint32(0) + ks[0], np.arange(B * V, dtype=np.uint32) + ks[1]]
    with np.errstate(over="ignore"):
        for i in range(5):
            for r in rot[i % 2]:
                x[0] = x[0] + x[1]
                x[1] = rotl(x[1], int(r))
                x[1] = x[0] ^ x[1]
            x[0] = x[0] + ks[(i + 1) % 3]
            x[1] = x[1] + ks[(i + 2) % 3] + np.uint32(i + 1)
    bits = x[0] ^ x[1]
    float_bits = (bits >> np.uint32(9)) | np.uint32(0x3F800000)
    floats = float_bits.view(np.float32) - np.float32(1.0)
    tiny = np.float32(np.finfo(np.float32).tiny)
    u = np.maximum(tiny, floats * (np.float32(1.0) - tiny) + tiny)
    g = (-np.log(-np.log(u))).astype(np.float32).reshape(B, V)
    g[:, 0] = _NEG_INF
    return g


_GUMBEL = _gumbel_constant()


def _sample_kernel(x_ref, g_ref, out_ref, m_scr, a_scr):
    i = pl.program_id(0)

    @pl.when(i == 0)
    def _init():
        m_scr[...] = jnp.full((B, 1), _NEG_INF, jnp.float32)
        a_scr[...] = jnp.zeros((B, 1), jnp.int32)

    s = x_ref[0] + g_ref[...]
    col = jax.lax.broadcasted_iota(jnp.int32, (B, VC), 1) + i * VC
    s = jnp.where(col < V, s, _NEG_INF)
    loc_max = jnp.max(s, axis=1, keepdims=True)
    cand = jnp.where(s == loc_max, col, V)
    loc_arg = jnp.min(cand, axis=1, keepdims=True)
    better = loc_max > m_scr[...]
    a_scr[...] = jnp.where(better, loc_arg, a_scr[...])
    m_scr[...] = jnp.maximum(loc_max, m_scr[...])

    @pl.when(i == NBLK - 1)
    def _fin():
        out_ref[...] = a_scr[...]


def kernel(decoder_out):
    return pl.pallas_call(
        _sample_kernel,
        grid=(NBLK,),
        in_specs=[
            pl.BlockSpec((1, B, VC), lambda i: (0, 0, i)),
            pl.BlockSpec((B, VC), lambda i: (0, i)),
        ],
        out_specs=pl.BlockSpec((B, 1), lambda i: (0, 0)),
        out_shape=jax.ShapeDtypeStruct((B, 1), jnp.int32),
        scratch_shapes=[
            pltpu.VMEM((B, 1), jnp.float32),
            pltpu.VMEM((B, 1), jnp.int32),
        ],
    )(decoder_out, _GUMBEL)


# speedup vs baseline: 2.6220x; 1.0149x over previous
import numpy as np

import jax
import jax.numpy as jnp
from jax.experimental import pallas as pl
from jax.experimental.pallas import tpu as pltpu

B = 128
V = 100000
VC = 12800
NBLK = (V + VC - 1) // VC

_NEG_INF = float("-inf")


def _gumbel_constant() -> np.ndarray:
    def rotl(x, d):
        return (x << np.uint32(d)) | (x >> np.uint32(32 - d))

    rot = [np.uint32([13, 15, 26, 6]), np.uint32([17, 29, 16, 24])]
    k1, k2 = np.uint32(0), np.uint32(42)
    ks = [k1, k2, np.uint32(k1 ^ k2 ^ np.uint32(0x1BD11BDA))]
    x = [np.uint32(0) + ks[0], np.arange(B * V, dtype=np.uint32) + ks[1]]
    with np.errstate(over="ignore"):
        for i in range(5):
            for r in rot[i % 2]:
                x[0] = x[0] + x[1]
                x[1] = rotl(x[1], int(r))
                x[1] = x[0] ^ x[1]
            x[0] = x[0] + ks[(i + 1) % 3]
            x[1] = x[1] + ks[(i + 2) % 3] + np.uint32(i + 1)
    bits = x[0] ^ x[1]
    float_bits = (bits >> np.uint32(9)) | np.uint32(0x3F800000)
    floats = float_bits.view(np.float32) - np.float32(1.0)
    tiny = np.float32(np.finfo(np.float32).tiny)
    u = np.maximum(tiny, floats * (np.float32(1.0) - tiny) + tiny)
    g = (-np.log(-np.log(u))).astype(np.float32).reshape(B, V)
    g[:, 0] = _NEG_INF
    return g


_GUMBEL = _gumbel_constant()


def _sample_kernel(x_ref, g_ref, out_ref, m_scr, a_scr):
    i = pl.program_id(0)

    @pl.when(i == 0)
    def _init():
        m_scr[...] = jnp.full((B, 1), _NEG_INF, jnp.float32)
        a_scr[...] = jnp.zeros((B, 1), jnp.int32)

    s = x_ref[0] + g_ref[...]
    col = jax.lax.broadcasted_iota(jnp.int32, (B, VC), 1) + i * VC
    s = jnp.where(col < V, s, _NEG_INF)
    loc_max = jnp.max(s, axis=1, keepdims=True)
    cand = jnp.where(s == loc_max, col, V)
    loc_arg = jnp.min(cand, axis=1, keepdims=True)
    better = loc_max > m_scr[...]
    a_scr[...] = jnp.where(better, loc_arg, a_scr[...])
    m_scr[...] = jnp.maximum(loc_max, m_scr[...])

    @pl.when(i == NBLK - 1)
    def _fin():
        out_ref[...] = a_scr[...]


def kernel(decoder_out):
    return pl.pallas_call(
        _sample_kernel,
        grid=(NBLK,),
        in_specs=[
            pl.BlockSpec((1, B, VC), lambda i: (0, 0, i)),
            pl.BlockSpec((B, VC), lambda i: (0, i)),
        ],
        out_specs=pl.BlockSpec((B, 1), lambda i: (0, 0)),
        out_shape=jax.ShapeDtypeStruct((B, 1), jnp.int32),
        scratch_shapes=[
            pltpu.VMEM((B, 1), jnp.float32),
            pltpu.VMEM((B, 1), jnp.int32),
        ],
    )(decoder_out, _GUMBEL)
